# baseline (speedup 1.0000x reference)
import functools

import jax
import jax.numpy as jnp
from jax import lax
from jax.experimental import pallas as pl
from jax.experimental.pallas import tpu as pltpu
from jax.experimental.pallas import tpu_sc as plsc

VOCAB = 1_000_000
D = 32
B = 4096
L = 200

NC = 2
NS = 16
NW = NC * NS
SW = B // NW

CS = 8
CHUNK_IDS = CS * L
G = 80
NG = CHUNK_IDS // G
NCHUNK = SW // CS

_mesh = plsc.VectorSubcoreMesh(core_axis_name="c", subcore_axis_name="s")


@functools.partial(
    pl.kernel,
    out_type=[
        jax.ShapeDtypeStruct((B * L, D), jnp.float32),
        jax.ShapeDtypeStruct((B, D), jnp.float32),
    ],
    mesh=_mesh,
    compiler_params=pltpu.CompilerParams(use_tc_tiling_on_sc=False),
    scratch_types=[
        pltpu.VMEM((CHUNK_IDS,), jnp.int32),
        pltpu.VMEM((CHUNK_IDS,), jnp.int32),
        pltpu.VMEM((CHUNK_IDS, D), jnp.float32),
        pltpu.VMEM((CHUNK_IDS, D), jnp.float32),
        pltpu.VMEM((CS, D), jnp.float32),
        pltpu.SemaphoreType.DMA,
        pltpu.SemaphoreType.DMA,
        pltpu.SemaphoreType.DMA,
        pltpu.SemaphoreType.DMA,
    ],
)
def _sc_embed(ids_hbm, table_hbm, seq_hbm, pool_hbm,
              idx0, idx1, rows0, rows1, pool_v, sg0, sg1, sw0, sw1):
    wid = lax.axis_index("s") * NC + lax.axis_index("c")
    idx = (idx0, idx1)
    rows = (rows0, rows1)
    sg = (sg0, sg1)
    sw = (sw0, sw1)

    def i_base(c):
        return (wid * SW + c * CS) * L

    def load_and_fire(c, b):
        pltpu.sync_copy(ids_hbm.at[pl.ds(i_base(c), CHUNK_IDS)], idx[b])
        for j in range(NG):
            pltpu.async_copy(
                table_hbm.at[idx[b].at[pl.ds(j * G, G)]],
                rows[b].at[pl.ds(j * G, G)],
                sg[b],
            )

    def drain_gathers(c, b):
        for j in range(NG):
            pltpu.make_async_copy(
                table_hbm.at[idx[b].at[pl.ds(j * G, G)]],
                rows[b].at[pl.ds(j * G, G)],
                sg[b],
            ).wait()

    def start_writeout(c, b):
        pltpu.async_copy(rows[b], seq_hbm.at[pl.ds(i_base(c), CHUNK_IDS)], sw[b])

    def wait_writeout(c, b):
        pltpu.make_async_copy(
            rows[b], seq_hbm.at[pl.ds(i_base(c), CHUNK_IDS)], sw[b]
        ).wait()

    def compute_pool(c, b):
        rows_b = rows[b]

        def sample_body(s, _):
            rb = s * L

            def row_body(i, carry):
                a0, a1 = carry
                r0 = rb + i * 8
                for r in range(8):
                    a0 = a0 + rows_b[r0 + r, pl.ds(0, 16)]
                    a1 = a1 + rows_b[r0 + r, pl.ds(16, 16)]
                return (a0, a1)

            z = jnp.zeros((16,), jnp.float32)
            a0, a1 = lax.fori_loop(0, L // 8, row_body, (z, z))
            pool_v[s, pl.ds(0, 16)] = a0 * (1.0 / L)
            pool_v[s, pl.ds(16, 16)] = a1 * (1.0 / L)
            return 0

        lax.fori_loop(0, CS, sample_body, 0)
        pltpu.sync_copy(pool_v, pool_hbm.at[pl.ds(wid * SW + c * CS, CS)])

    load_and_fire(0, 0)
    for c in range(NCHUNK):
        b = c & 1
        nb = 1 - b
        if c + 1 < NCHUNK:
            if c >= 1:
                wait_writeout(c - 1, nb)
            load_and_fire(c + 1, nb)
        drain_gathers(c, b)
        start_writeout(c, b)
        compute_pool(c, b)
    wait_writeout(NCHUNK - 2, (NCHUNK - 2) & 1)
    wait_writeout(NCHUNK - 1, (NCHUNK - 1) & 1)


def kernel(input_ids, embeddings):
    ids_flat = input_ids.reshape(-1).astype(jnp.int32)
    seq_flat, pooled = _sc_embed(ids_flat, embeddings)
    return seq_flat.reshape(B, L, D), pooled
